# baseline (speedup 1.0000x reference)
import functools

import jax
import jax.numpy as jnp
from jax import lax
from jax.experimental import pallas as pl
from jax.experimental.pallas import tpu as pltpu
from jax.experimental.pallas import tpu_sc as plsc

_B = 8
_H = 16
_P = 64
_D = 64
_NPOS = 6
_ROWS_PER_J = _P * _P
_HALF_ROWS = _ROWS_PER_J // 2
_NC = 2
_NS = 16
_NW = _NC * _NS
_GATHER = 128
_ROUND = 512
_RG = _ROUND // _GATHER
_L = 16
_REP = 64


_mesh = plsc.VectorSubcoreMesh(core_axis_name="c", subcore_axis_name="s")


def _encode_half(half):
    @functools.partial(
        pl.kernel,
        mesh=_mesh,
        out_type=jax.ShapeDtypeStruct((_B * _HALF_ROWS, 2 * _D), jnp.float32),
        scratch_types=[
            pltpu.VMEM((_RG, _GATHER), jnp.int32),
            pltpu.VMEM((_RG, _GATHER), jnp.int32),
            pltpu.VMEM((_ROUND, 2 * _D), jnp.float32),
            pltpu.SemaphoreType.DMA,
        ],
    )
    def _encode(idx_hbm, tp_hbm, g_hbm, idx_v, sidx_v, rows_v, gsem):
        wid = lax.axis_index("s") * _NC + lax.axis_index("c")
        j = wid % _B
        q = wid // _B

        row0 = pl.multiple_of(j * 32 + half * 16 + q * _RG, _RG)
        pltpu.sync_copy(idx_hbm.at[pl.ds(row0, _RG)], idx_v)

        lane = lax.iota(jnp.int32, _L)
        for i in range(_RG):
            for g in range(_GATHER // _L):
                rep = (g * _L) % _REP
                salt = _NPOS * (lane + rep)
                sidx_v[i, pl.ds(g * _L, _L)] = (
                    idx_v[i, pl.ds(g * _L, _L)] + salt
                )

        gathers = [
            pltpu.async_copy(
                tp_hbm.at[sidx_v.at[i]],
                rows_v.at[pl.ds(i * _GATHER, _GATHER)],
                gsem,
            )
            for i in range(_RG)
        ]
        for g in gathers:
            g.wait()

        base = pl.multiple_of(j * _HALF_ROWS + q * _ROUND, _ROUND)
        pltpu.sync_copy(rows_v, g_hbm.at[pl.ds(base, _ROUND)])

    return _encode


_encode_a = _encode_half(0)
_encode_b = _encode_half(1)

_PC = 4
_HPC = _P // 2 // _PC


def _bcast_a_body(g_ref, out_ref):
    g = g_ref[...]
    g4 = g[:, :, :_D].reshape(_B, _PC, _P, _D)
    out6 = jnp.broadcast_to(g4[None, None], (_B, 2, _B, _PC, _P, _D))
    out_ref[...] = out6.reshape(_B, _H, _PC, _P, _D)


def _bcast_b_body(g_ref, prev_ref, out_ref):
    del prev_ref
    _bcast_a_body(g_ref, out_ref)


_OUT5 = jax.ShapeDtypeStruct((_B, _H, _P, _P, _D), jnp.float32)

_broadcast_a = pl.pallas_call(
    _bcast_a_body,
    grid=(_HPC,),
    in_specs=[pl.BlockSpec((_B, _PC * _P, 2 * _D), lambda c: (0, c, 0))],
    out_specs=pl.BlockSpec((_B, _H, _PC, _P, _D), lambda c: (0, 0, c, 0, 0)),
    out_shape=_OUT5,
)

_broadcast_b = pl.pallas_call(
    _bcast_b_body,
    grid=(_HPC,),
    in_specs=[
        pl.BlockSpec((_B, _PC * _P, 2 * _D), lambda c: (0, c, 0)),
        pl.BlockSpec(memory_space=pltpu.HBM),
    ],
    out_specs=pl.BlockSpec(
        (_B, _H, _PC, _P, _D), lambda c: (0, 0, c + _HPC, 0, 0)
    ),
    out_shape=_OUT5,
    input_output_aliases={1: 0},
)


def kernel(src_seq, structure_emb):
    idx2d = src_seq.reshape(_B * _ROWS_PER_J // 128, 128).astype(jnp.int32)
    emb = structure_emb.astype(jnp.float32)
    tp = jnp.tile(jnp.pad(emb, ((0, 0), (0, 2 * _D - _D))), (_REP, 1))
    ga = _encode_a(idx2d, tp)
    gb = _encode_b(idx2d, tp)
    ga3 = ga.reshape(_B, _HALF_ROWS, 2 * _D)
    gb3 = gb.reshape(_B, _HALF_ROWS, 2 * _D)
    half1 = _broadcast_a(ga3)
    return _broadcast_b(gb3, half1)
